# baseline (speedup 1.0000x reference)
import jax
import jax.numpy as jnp
from jax import lax
from jax.experimental import pallas as pl
from jax.experimental.pallas import tpu as pltpu
from jax.experimental.pallas import tpu_sc as plsc

NUM_ENT = 1000000
NUM_REL = 1000
D = 64
B = 16384
NC = 2
NS = 16
L = 16
NW = NC * NS
BPW = B // NW
CHUNK = 128
NCHUNK = BPW // CHUNK
GROUPS = CHUNK // L
DL = D // L

NB = (NUM_ENT + 127) // 128
SUP = 4
NSUP = (NB + SUP - 1) // SUP
NSW = (NSUP + NW - 1) // NW
NBW = NSW * SUP
NBW_PAD = 256
KBIN = 32
TAIL_SUP = NSUP - 1
TAIL_BLK = NB - 1
TAIL_BASE = TAIL_BLK * 128
SCAN = B
COMP = 1024
SB_ROWS = 256

_LANE = None


def _lane():
    return lax.broadcasted_iota(jnp.int32, (L,), 0)


def _scalar(vec, j=0):
    return jnp.squeeze(lax.slice(vec, (j,), (j + 1,)))


def _splat(x):
    return jnp.broadcast_to(x, (L,))


def _take16(x, idx):
    dn = lax.GatherDimensionNumbers(offset_dims=(), collapsed_slice_dims=(0,),
                                    start_index_map=(0,))
    return lax.gather(x, idx[:, None], dn, (1,),
                      mode=lax.GatherScatterMode.PROMISE_IN_BOUNDS)


def _extract_body(heads_hbm, tails_hbm, entT_hbm, tail128_hbm,
                  stage_hbm,
                  scan_v, comp_v, counts_v, bins_v, blk_v, sb_v, slot_v,
                  sem_b0, sem_b1, sem_s):
    wid = lax.axis_index("s") * NC + lax.axis_index("c")
    blo = wid * NBW
    nbw = jnp.minimum(NBW, jnp.maximum(NB - blo, 0))
    lane = _lane()
    lane0 = lane == 0
    lane15 = lane == L - 1

    for k in range(NBW_PAD // L):
        counts_v[pl.ds(k * L, L)] = jnp.zeros((L,), jnp.int32)

    def scan_chunk(src_hbm, slot_off):
        pltpu.sync_copy(src_hbm, scan_v)

        def compress(k, off):
            for u in range(4):
                iv = scan_v[pl.ds((k * 4 + u) * L, L)]
                b = iv >> 7
                m = (b >= blo) & (b < blo + nbw)
                blc = jnp.where(m, b - blo, 0)
                slot = slot_off + (k * 4 + u) * L + lane
                val = (((slot << 7) | (iv & 127)) << 8) | blc
                plsc.store_compressed(comp_v.at[pl.ds(off, L)], val, mask=m)
                npop = plsc.all_reduce_population_count(m)
                off = jnp.minimum(off + _scalar(npop), COMP - L)
            return off

        nmatch = lax.fori_loop(0, B // L // 4, compress, jnp.int32(0))

        def heavy(k, _):
            rem = nmatch - k * L
            vs0 = comp_v[pl.ds(k * L, L)]
            m = lane < rem
            blc = vs0 & 255
            key = jnp.where(m, blc, jnp.int32(0x7FFFFFFF))
            ks, vs = plsc.sort_key_val(key, vs0)
            ms = ks != jnp.int32(0x7FFFFFFF)
            blcs = vs & 255
            entry = lax.shift_right_logical(vs, 8)
            prev = _take16(ks, jnp.maximum(lane - 1, 0))
            nxt = _take16(ks, jnp.minimum(lane + 1, L - 1))
            startm = (ks != prev) | lane0
            lastm = (ks != nxt) | lane15
            rsp = plsc.cummax(jnp.where(startm, lane, 0))
            occ = lane - rsp
            base = plsc.load_gather(counts_v, [blcs], mask=ms)
            pos = jnp.minimum(base + occ, KBIN - 1)
            plsc.store_scatter(bins_v, [blcs * KBIN + pos], entry,
                               mask=ms)
            plsc.store_scatter(counts_v, [blcs], pos + 1,
                               mask=ms & lastm)
            return 0

        lax.fori_loop(0, (nmatch + L - 1) // L, heavy, 0)

    scan_chunk(heads_hbm, 0)
    scan_chunk(tails_hbm, B)

    neg1 = jnp.full((L,), -1, jnp.int32)
    for g2 in range(2):
        for k in range(CHUNK // L):
            slot_v[g2, pl.ds(k * L, L)] = neg1

    sup0 = wid * NSW
    nsw = jnp.minimum(NSW, jnp.maximum(NSUP - sup0, 0))

    def issue(sup, buf_slot, sem):
        @pl.when(sup == TAIL_SUP)
        def _():
            pltpu.async_copy(tail128_hbm,
                             blk_v.at[pl.ds(buf_slot * D, D),
                                      pl.ds(0, 128)], sem)

        @pl.when(sup != TAIL_SUP)
        def _():
            start = pl.multiple_of(sup * (SUP * 128), SUP * 128)
            pltpu.async_copy(entT_hbm.at[:, pl.ds(start, SUP * 128)],
                             blk_v.at[pl.ds(buf_slot * D, D), :], sem)

    def drain(sup, buf_slot, sem):
        @pl.when(sup == TAIL_SUP)
        def _():
            pltpu.make_async_copy(
                entT_hbm.at[:, pl.ds(0, 128)],
                blk_v.at[pl.ds(buf_slot * D, D), pl.ds(0, 128)],
                sem).wait()

        @pl.when(sup != TAIL_SUP)
        def _():
            pltpu.make_async_copy(
                entT_hbm.at[:, pl.ds(0, SUP * 128)],
                blk_v.at[pl.ds(buf_slot * D, D), :], sem).wait()

    def process_block(bl, buf_slot, q, sbc0):
        cntv = plsc.load_gather(counts_v, [_splat(bl)])
        cnt = _scalar(cntv)

        def entry_body(j2, sbc):
            ev = plsc.load_gather(bins_v, [_splat(bl * KBIN + j2)])
            e = _scalar(ev)
            c = (e & 127) + q * 128
            slot = lax.shift_right_logical(e, 7)
            r = sbc & (CHUNK - 1)
            grp = lax.shift_right_logical(sbc, 7) & 1
            row = grp * CHUNK + r
            for d in range(DL):
                vals = plsc.load_gather(
                    blk_v, [_splat(buf_slot * D + d * L) + lane, _splat(c)])
                sb_v[row, pl.ds(d * L, L)] = vals
            plsc.store_scatter(slot_v, [_splat(grp), _splat(r)],
                               _splat(slot), mask=lane0)

            @pl.when(r == CHUNK - 1)
            def _():
                @pl.when(sbc >= 2 * CHUNK - 1)
                def _():
                    pltpu.make_async_copy(
                        stage_hbm.at[pl.ds(0, CHUNK), :],
                        sb_v.at[pl.ds(0, CHUNK), :], sem_s).wait()
                    for k in range(CHUNK // L):
                        slot_v[1 - grp, pl.ds(k * L, L)] = neg1

                pltpu.async_copy(
                    sb_v.at[pl.ds(grp * CHUNK, CHUNK), :],
                    stage_hbm.at[plsc.Indices(slot_v.at[grp],
                                              ignored_value=-1)],
                    sem_s)

            return sbc + 1

        return lax.fori_loop(0, cnt, entry_body, sbc0)

    def process_sup(sp, buf_slot, sbc):
        for q in range(SUP):
            bl = sp * SUP + q
            sbc = lax.cond(bl < nbw,
                           lambda s, bl=bl, q=q: process_block(
                               bl, buf_slot, q, s),
                           lambda s: s, sbc)
        return sbc

    @pl.when(nsw > 0)
    def _():
        issue(sup0, 0, sem_b0)

    npair = (nsw + 1) // 2

    def pair_body(p, sbc):
        sp0 = p * 2
        sp1 = sp0 + 1
        drain(sup0 + sp0, 0, sem_b0)

        @pl.when(sp1 < nsw)
        def _():
            issue(sup0 + sp1, 1, sem_b1)

        sbc = process_sup(sp0, 0, sbc)

        @pl.when(sp1 < nsw)
        def _():
            drain(sup0 + sp1, 1, sem_b1)

        @pl.when(sp0 + 2 < nsw)
        def _():
            issue(sup0 + sp0 + 2, 0, sem_b0)

        return lax.cond(sp1 < nsw,
                        lambda s: process_sup(sp1, 1, s),
                        lambda s: s, sbc)

    sb_cnt = lax.fori_loop(0, npair, pair_body, jnp.int32(0))

    @pl.when(sb_cnt >= CHUNK)
    def _():
        pltpu.make_async_copy(stage_hbm.at[pl.ds(0, CHUNK), :],
                              sb_v.at[pl.ds(0, CHUNK), :], sem_s).wait()

    @pl.when((sb_cnt & (CHUNK - 1)) != 0)
    def _():
        grp = lax.shift_right_logical(sb_cnt, 7) & 1
        cp = pltpu.async_copy(
            sb_v.at[pl.ds(grp * CHUNK, CHUNK), :],
            stage_hbm.at[plsc.Indices(slot_v.at[grp], ignored_value=-1)],
            sem_s)
        cp.wait()


def _score_body(rels_hbm, stage_hbm, rel128_hbm, out_hbm,
                ir_v, h_v, t_v, rn_v, scores_v, sem):
    wid = lax.axis_index("s") * NC + lax.axis_index("c")
    base = wid * BPW
    lane = _lane()

    def do_chunk(c, _):
        off = base + c * CHUNK
        pltpu.sync_copy(rels_hbm.at[pl.ds(off, CHUNK)], ir_v)
        cp_h = pltpu.async_copy(stage_hbm.at[pl.ds(off, CHUNK), :], h_v,
                                sem)
        cp_t = pltpu.async_copy(stage_hbm.at[pl.ds(B + off, CHUNK), :], t_v,
                                sem)
        cp_r = pltpu.async_copy(rel128_hbm.at[ir_v], rn_v, sem)
        cp_h.wait()
        cp_t.wait()
        cp_r.wait()

        def do_group(g, _):
            vec = jnp.zeros((L,), jnp.float32)
            for j in range(L):
                row = g * L + j
                h = [h_v[row, pl.ds(d * L, L)] for d in range(DL)]
                t = [t_v[row, pl.ds(d * L, L)] for d in range(DL)]
                r = [rn_v[row, pl.ds(d * L, L)] for d in range(DL)]
                n = [rn_v[row, pl.ds(D + d * L, L)] for d in range(DL)]
                dh = jnp.sum((h[0] * n[0] + h[1] * n[1])
                             + (h[2] * n[2] + h[3] * n[3]))
                dt = jnp.sum((t[0] * n[0] + t[1] * n[1])
                             + (t[2] * n[2] + t[3] * n[3]))
                parts = [jnp.abs((h[d] - t[d]) + r[d] + (dt - dh) * n[d])
                         for d in range(DL)]
                s = jnp.sum((parts[0] + parts[1]) + (parts[2] + parts[3]))
                vec = jnp.where(lane == j, s, vec)
            scores_v[pl.ds(c * CHUNK + g * L, L)] = vec
            return 0

        lax.fori_loop(0, GROUPS, do_group, 0)
        return 0

    lax.fori_loop(0, NCHUNK, do_chunk, 0)
    pltpu.sync_copy(scores_v, out_hbm.at[pl.ds(base, BPW)])


@jax.jit
def kernel(heads, rels, tails, ent_embs, rel_embs, norm_vector):
    entT = ent_embs.T
    tail128 = jnp.concatenate(
        [ent_embs[TAIL_BASE:].T,
         jnp.zeros((D, 128 - (NUM_ENT - TAIL_BASE)), jnp.float32)], axis=1)
    rel128 = jnp.concatenate([rel_embs, norm_vector], axis=1)
    mesh = plsc.VectorSubcoreMesh(core_axis_name="c", subcore_axis_name="s",
                                  num_cores=NC, num_subcores=NS)
    cp = pltpu.CompilerParams(needs_layout_passes=False,
                              use_tc_tiling_on_sc=True)

    stage = pl.kernel(
        _extract_body,
        out_type=jax.ShapeDtypeStruct((2 * B, 128), jnp.float32),
        mesh=mesh,
        compiler_params=cp,
        scratch_types=[
            pltpu.VMEM((SCAN,), jnp.int32),
            pltpu.VMEM((COMP,), jnp.int32),
            pltpu.VMEM((NBW_PAD,), jnp.int32),
            pltpu.VMEM((NBW_PAD * KBIN,), jnp.int32),
            pltpu.VMEM((2 * D, SUP * 128), jnp.float32),
            pltpu.VMEM((SB_ROWS, 128), jnp.float32),
            pltpu.VMEM((2, CHUNK), jnp.int32),
            pltpu.SemaphoreType.DMA,
            pltpu.SemaphoreType.DMA,
            pltpu.SemaphoreType.DMA,
        ],
    )(heads, tails, entT, tail128)

    scores = pl.kernel(
        _score_body,
        out_type=jax.ShapeDtypeStruct((B,), jnp.float32),
        mesh=mesh,
        compiler_params=cp,
        scratch_types=[
            pltpu.VMEM((CHUNK,), jnp.int32),
            pltpu.VMEM((CHUNK, 128), jnp.float32),
            pltpu.VMEM((CHUNK, 128), jnp.float32),
            pltpu.VMEM((CHUNK, 128), jnp.float32),
            pltpu.VMEM((BPW,), jnp.float32),
            pltpu.SemaphoreType.DMA,
        ],
    )(rels, stage, rel128)
    return scores


# speedup vs baseline: 1.0001x; 1.0001x over previous
import functools

import jax
import jax.numpy as jnp
from jax import lax
from jax.experimental import pallas as pl
from jax.experimental.pallas import tpu as pltpu
from jax.experimental.pallas import tpu_sc as plsc

NUM_ENT = 1000000
NUM_REL = 1000
D = 64
B = 16384
NC = 2
NS = 16
L = 16
NW = NC * NS
BPW = B // NW
CHUNK = 128
NCHUNK = BPW // CHUNK
GROUPS = CHUNK // L
DL = D // L

NB = (NUM_ENT + 127) // 128
SUP = 4
NSUP = (NB + SUP - 1) // SUP
NSW = (NSUP + NW - 1) // NW
NBW = NSW * SUP
NBW_PAD = 256
KBIN = 32
TAIL_SUP = NSUP - 1
TAIL_BLK = NB - 1
TAIL_BASE = TAIL_BLK * 128
SCAN = B
COMP = 1024
SB_ROWS = 256

_LANE = None


def _lane():
    return lax.broadcasted_iota(jnp.int32, (L,), 0)


def _scalar(vec, j=0):
    return jnp.squeeze(lax.slice(vec, (j,), (j + 1,)))


def _splat(x):
    return jnp.broadcast_to(x, (L,))


def _take16(x, idx):
    dn = lax.GatherDimensionNumbers(offset_dims=(), collapsed_slice_dims=(0,),
                                    start_index_map=(0,))
    return lax.gather(x, idx[:, None], dn, (1,),
                      mode=lax.GatherScatterMode.PROMISE_IN_BOUNDS)


def _extract_body(heads_hbm, tails_hbm, entT_hbm, tail128_hbm,
                  stage_hbm,
                  scan_v, comp_v, counts_v, bins_v, blk_v, sb_v, slot_v,
                  sem_b0, sem_b1, sem_s):
    wid = lax.axis_index("s") * NC + lax.axis_index("c")
    blo = wid * NBW
    nbw = jnp.minimum(NBW, jnp.maximum(NB - blo, 0))
    lane = _lane()
    lane0 = lane == 0
    lane15 = lane == L - 1

    for k in range(NBW_PAD // L):
        counts_v[pl.ds(k * L, L)] = jnp.zeros((L,), jnp.int32)

    def scan_chunk(src_hbm, slot_off):
        pltpu.sync_copy(src_hbm, scan_v)

        NQ = B // L // 4

        def compress(k, offs):
            new = []
            for u in range(4):
                off = offs[u]
                iv = scan_v[pl.ds((u * NQ + k) * L, L)]
                b = iv >> 7
                m = (b >= blo) & (b < blo + nbw)
                blc = jnp.where(m, b - blo, 0)
                slot = slot_off + (u * NQ + k) * L + lane
                val = (((slot << 7) | (iv & 127)) << 8) | blc
                plsc.store_compressed(
                    comp_v.at[pl.ds(u * (COMP // 4) + off, L)], val, mask=m)
                npop = plsc.all_reduce_population_count(m)
                new.append(jnp.minimum(off + _scalar(npop), COMP // 4 - L))
            return tuple(new)

        offs = lax.fori_loop(0, NQ, compress,
                             (jnp.int32(0),) * 4)

        def heavy_seg(u, nmatch, k, _):
            rem = nmatch - k * L
            vs0 = comp_v[pl.ds(u * (COMP // 4) + k * L, L)]
            m = lane < rem
            blc = vs0 & 255
            key = jnp.where(m, blc, jnp.int32(0x7FFFFFFF))
            ks, vs = plsc.sort_key_val(key, vs0)
            ms = ks != jnp.int32(0x7FFFFFFF)
            blcs = vs & 255
            entry = lax.shift_right_logical(vs, 8)
            prev = _take16(ks, jnp.maximum(lane - 1, 0))
            nxt = _take16(ks, jnp.minimum(lane + 1, L - 1))
            startm = (ks != prev) | lane0
            lastm = (ks != nxt) | lane15
            rsp = plsc.cummax(jnp.where(startm, lane, 0))
            occ = lane - rsp
            base = plsc.load_gather(counts_v, [blcs], mask=ms)
            pos = jnp.minimum(base + occ, KBIN - 1)
            plsc.store_scatter(bins_v, [blcs * KBIN + pos], entry,
                               mask=ms)
            plsc.store_scatter(counts_v, [blcs], pos + 1,
                               mask=ms & lastm)
            return 0

        for u in range(4):
            lax.fori_loop(0, (offs[u] + L - 1) // L,
                          functools.partial(heavy_seg, u, offs[u]), 0)

    scan_chunk(heads_hbm, 0)
    scan_chunk(tails_hbm, B)

    neg1 = jnp.full((L,), -1, jnp.int32)
    for g2 in range(2):
        for k in range(CHUNK // L):
            slot_v[g2, pl.ds(k * L, L)] = neg1

    sup0 = wid * NSW
    nsw = jnp.minimum(NSW, jnp.maximum(NSUP - sup0, 0))

    def issue(sup, buf_slot, sem):
        @pl.when(sup == TAIL_SUP)
        def _():
            pltpu.async_copy(tail128_hbm,
                             blk_v.at[pl.ds(buf_slot * D, D),
                                      pl.ds(0, 128)], sem)

        @pl.when(sup != TAIL_SUP)
        def _():
            start = pl.multiple_of(sup * (SUP * 128), SUP * 128)
            pltpu.async_copy(entT_hbm.at[:, pl.ds(start, SUP * 128)],
                             blk_v.at[pl.ds(buf_slot * D, D), :], sem)

    def drain(sup, buf_slot, sem):
        @pl.when(sup == TAIL_SUP)
        def _():
            pltpu.make_async_copy(
                entT_hbm.at[:, pl.ds(0, 128)],
                blk_v.at[pl.ds(buf_slot * D, D), pl.ds(0, 128)],
                sem).wait()

        @pl.when(sup != TAIL_SUP)
        def _():
            pltpu.make_async_copy(
                entT_hbm.at[:, pl.ds(0, SUP * 128)],
                blk_v.at[pl.ds(buf_slot * D, D), :], sem).wait()

    def process_block(bl, buf_slot, q, sbc0):
        cntv = plsc.load_gather(counts_v, [_splat(bl)])
        cnt = _scalar(cntv)

        def entry_body(j2, sbc):
            ev = plsc.load_gather(bins_v, [_splat(bl * KBIN + j2)])
            e = _scalar(ev)
            c = (e & 127) + q * 128
            slot = lax.shift_right_logical(e, 7)
            r = sbc & (CHUNK - 1)
            grp = lax.shift_right_logical(sbc, 7) & 1
            row = grp * CHUNK + r
            for d in range(DL):
                vals = plsc.load_gather(
                    blk_v, [_splat(buf_slot * D + d * L) + lane, _splat(c)])
                sb_v[row, pl.ds(d * L, L)] = vals
            plsc.store_scatter(slot_v, [_splat(grp), _splat(r)],
                               _splat(slot), mask=lane0)

            @pl.when(r == CHUNK - 1)
            def _():
                @pl.when(sbc >= 2 * CHUNK - 1)
                def _():
                    pltpu.make_async_copy(
                        stage_hbm.at[pl.ds(0, CHUNK), :],
                        sb_v.at[pl.ds(0, CHUNK), :], sem_s).wait()
                    for k in range(CHUNK // L):
                        slot_v[1 - grp, pl.ds(k * L, L)] = neg1

                pltpu.async_copy(
                    sb_v.at[pl.ds(grp * CHUNK, CHUNK), :],
                    stage_hbm.at[plsc.Indices(slot_v.at[grp],
                                              ignored_value=-1)],
                    sem_s)

            return sbc + 1

        return lax.fori_loop(0, cnt, entry_body, sbc0)

    def process_sup(sp, buf_slot, sbc):
        for q in range(SUP):
            bl = sp * SUP + q
            sbc = lax.cond(bl < nbw,
                           lambda s, bl=bl, q=q: process_block(
                               bl, buf_slot, q, s),
                           lambda s: s, sbc)
        return sbc

    @pl.when(nsw > 0)
    def _():
        issue(sup0, 0, sem_b0)

    npair = (nsw + 1) // 2

    def pair_body(p, sbc):
        sp0 = p * 2
        sp1 = sp0 + 1
        drain(sup0 + sp0, 0, sem_b0)

        @pl.when(sp1 < nsw)
        def _():
            issue(sup0 + sp1, 1, sem_b1)

        sbc = process_sup(sp0, 0, sbc)

        @pl.when(sp1 < nsw)
        def _():
            drain(sup0 + sp1, 1, sem_b1)

        @pl.when(sp0 + 2 < nsw)
        def _():
            issue(sup0 + sp0 + 2, 0, sem_b0)

        return lax.cond(sp1 < nsw,
                        lambda s: process_sup(sp1, 1, s),
                        lambda s: s, sbc)

    sb_cnt = lax.fori_loop(0, npair, pair_body, jnp.int32(0))

    @pl.when(sb_cnt >= CHUNK)
    def _():
        pltpu.make_async_copy(stage_hbm.at[pl.ds(0, CHUNK), :],
                              sb_v.at[pl.ds(0, CHUNK), :], sem_s).wait()

    @pl.when((sb_cnt & (CHUNK - 1)) != 0)
    def _():
        grp = lax.shift_right_logical(sb_cnt, 7) & 1
        cp = pltpu.async_copy(
            sb_v.at[pl.ds(grp * CHUNK, CHUNK), :],
            stage_hbm.at[plsc.Indices(slot_v.at[grp], ignored_value=-1)],
            sem_s)
        cp.wait()


def _score_body(rels_hbm, stage_hbm, rel128_hbm, out_hbm,
                ir_v, h_v, t_v, rn_v, scores_v, sem):
    wid = lax.axis_index("s") * NC + lax.axis_index("c")
    base = wid * BPW
    lane = _lane()

    def do_chunk(c, _):
        off = base + c * CHUNK
        pltpu.sync_copy(rels_hbm.at[pl.ds(off, CHUNK)], ir_v)
        cp_h = pltpu.async_copy(stage_hbm.at[pl.ds(off, CHUNK), :], h_v,
                                sem)
        cp_t = pltpu.async_copy(stage_hbm.at[pl.ds(B + off, CHUNK), :], t_v,
                                sem)
        cp_r = pltpu.async_copy(rel128_hbm.at[ir_v], rn_v, sem)
        cp_h.wait()
        cp_t.wait()
        cp_r.wait()

        def do_group(g, _):
            vec = jnp.zeros((L,), jnp.float32)
            for j in range(L):
                row = g * L + j
                h = [h_v[row, pl.ds(d * L, L)] for d in range(DL)]
                t = [t_v[row, pl.ds(d * L, L)] for d in range(DL)]
                r = [rn_v[row, pl.ds(d * L, L)] for d in range(DL)]
                n = [rn_v[row, pl.ds(D + d * L, L)] for d in range(DL)]
                dh = jnp.sum((h[0] * n[0] + h[1] * n[1])
                             + (h[2] * n[2] + h[3] * n[3]))
                dt = jnp.sum((t[0] * n[0] + t[1] * n[1])
                             + (t[2] * n[2] + t[3] * n[3]))
                parts = [jnp.abs((h[d] - t[d]) + r[d] + (dt - dh) * n[d])
                         for d in range(DL)]
                s = jnp.sum((parts[0] + parts[1]) + (parts[2] + parts[3]))
                vec = jnp.where(lane == j, s, vec)
            scores_v[pl.ds(c * CHUNK + g * L, L)] = vec
            return 0

        lax.fori_loop(0, GROUPS, do_group, 0)
        return 0

    lax.fori_loop(0, NCHUNK, do_chunk, 0)
    pltpu.sync_copy(scores_v, out_hbm.at[pl.ds(base, BPW)])


@jax.jit
def kernel(heads, rels, tails, ent_embs, rel_embs, norm_vector):
    entT = ent_embs.T
    tail128 = jnp.concatenate(
        [ent_embs[TAIL_BASE:].T,
         jnp.zeros((D, 128 - (NUM_ENT - TAIL_BASE)), jnp.float32)], axis=1)
    rel128 = jnp.concatenate([rel_embs, norm_vector], axis=1)
    mesh = plsc.VectorSubcoreMesh(core_axis_name="c", subcore_axis_name="s",
                                  num_cores=NC, num_subcores=NS)
    cp = pltpu.CompilerParams(needs_layout_passes=False,
                              use_tc_tiling_on_sc=True)

    stage = pl.kernel(
        _extract_body,
        out_type=jax.ShapeDtypeStruct((2 * B, 128), jnp.float32),
        mesh=mesh,
        compiler_params=cp,
        scratch_types=[
            pltpu.VMEM((SCAN,), jnp.int32),
            pltpu.VMEM((COMP,), jnp.int32),
            pltpu.VMEM((NBW_PAD,), jnp.int32),
            pltpu.VMEM((NBW_PAD * KBIN,), jnp.int32),
            pltpu.VMEM((2 * D, SUP * 128), jnp.float32),
            pltpu.VMEM((SB_ROWS, 128), jnp.float32),
            pltpu.VMEM((2, CHUNK), jnp.int32),
            pltpu.SemaphoreType.DMA,
            pltpu.SemaphoreType.DMA,
            pltpu.SemaphoreType.DMA,
        ],
    )(heads, tails, entT, tail128)

    scores = pl.kernel(
        _score_body,
        out_type=jax.ShapeDtypeStruct((B,), jnp.float32),
        mesh=mesh,
        compiler_params=cp,
        scratch_types=[
            pltpu.VMEM((CHUNK,), jnp.int32),
            pltpu.VMEM((CHUNK, 128), jnp.float32),
            pltpu.VMEM((CHUNK, 128), jnp.float32),
            pltpu.VMEM((CHUNK, 128), jnp.float32),
            pltpu.VMEM((BPW,), jnp.float32),
            pltpu.SemaphoreType.DMA,
        ],
    )(rels, stage, rel128)
    return scores


# speedup vs baseline: 1.0083x; 1.0082x over previous
import jax
import jax.numpy as jnp
from jax import lax
from jax.experimental import pallas as pl
from jax.experimental.pallas import tpu as pltpu
from jax.experimental.pallas import tpu_sc as plsc

NUM_ENT = 1000000
NUM_REL = 1000
D = 64
B = 16384
NC = 2
NS = 16
L = 16
NW = NC * NS
BPW = B // NW
CHUNK = 128
NCHUNK = BPW // CHUNK
GROUPS = CHUNK // L
DL = D // L

NB = (NUM_ENT + 127) // 128
SUP = 4
NSUP = (NB + SUP - 1) // SUP
NSW = (NSUP + NW - 1) // NW
NBW = NSW * SUP
NBW_PAD = 256
KBIN = 32
TAIL_SUP = NSUP - 1
TAIL_BLK = NB - 1
TAIL_BASE = TAIL_BLK * 128
SCAN = B
COMP = 1024
SB_ROWS = 256

_LANE = None


def _lane():
    return lax.broadcasted_iota(jnp.int32, (L,), 0)


def _scalar(vec, j=0):
    return jnp.squeeze(lax.slice(vec, (j,), (j + 1,)))


def _splat(x):
    return jnp.broadcast_to(x, (L,))


def _take16(x, idx):
    dn = lax.GatherDimensionNumbers(offset_dims=(), collapsed_slice_dims=(0,),
                                    start_index_map=(0,))
    return lax.gather(x, idx[:, None], dn, (1,),
                      mode=lax.GatherScatterMode.PROMISE_IN_BOUNDS)


def _extract_body(heads_hbm, tails_hbm, entT_hbm, tail128_hbm,
                  stage_hbm,
                  scan_v, comp_v, counts_v, bins_v, blk_v, sb_v, slot_v,
                  sem_b0, sem_b1, sem_s):
    wid = lax.axis_index("s") * NC + lax.axis_index("c")
    blo = wid * NBW
    nbw = jnp.minimum(NBW, jnp.maximum(NB - blo, 0))
    lane = _lane()
    lane0 = lane == 0
    lane15 = lane == L - 1

    for k in range(NBW_PAD // L):
        counts_v[pl.ds(k * L, L)] = jnp.zeros((L,), jnp.int32)

    def scan_chunk(src_hbm, slot_off):
        pltpu.sync_copy(src_hbm, scan_v)

        def compress(k, off):
            iv = scan_v[pl.ds(k * L, L)]
            b = iv >> 7
            m = (b >= blo) & (b < blo + nbw)
            blc = jnp.where(m, b - blo, 0)
            slot = slot_off + k * L + lane
            val = (((slot << 7) | (iv & 127)) << 8) | blc
            plsc.store_compressed(comp_v.at[pl.ds(off, L)], val, mask=m)
            npop = plsc.all_reduce_population_count(m)
            return jnp.minimum(off + _scalar(npop), COMP - L)

        nmatch = lax.fori_loop(0, B // L, compress, jnp.int32(0))

        def heavy(k, _):
            rem = nmatch - k * L
            vs0 = comp_v[pl.ds(k * L, L)]
            m = lane < rem
            blc = vs0 & 255
            key = jnp.where(m, blc, jnp.int32(0x7FFFFFFF))
            ks, vs = plsc.sort_key_val(key, vs0)
            ms = ks != jnp.int32(0x7FFFFFFF)
            blcs = vs & 255
            entry = lax.shift_right_logical(vs, 8)
            prev = _take16(ks, jnp.maximum(lane - 1, 0))
            nxt = _take16(ks, jnp.minimum(lane + 1, L - 1))
            startm = (ks != prev) | lane0
            lastm = (ks != nxt) | lane15
            rsp = plsc.cummax(jnp.where(startm, lane, 0))
            occ = lane - rsp
            base = plsc.load_gather(counts_v, [blcs], mask=ms)
            pos = jnp.minimum(base + occ, KBIN - 1)
            plsc.store_scatter(bins_v, [blcs * KBIN + pos], entry,
                               mask=ms)
            plsc.store_scatter(counts_v, [blcs], pos + 1,
                               mask=ms & lastm)
            return 0

        lax.fori_loop(0, (nmatch + L - 1) // L, heavy, 0)

    scan_chunk(heads_hbm, 0)
    scan_chunk(tails_hbm, B)

    neg1 = jnp.full((L,), -1, jnp.int32)
    for g2 in range(2):
        for k in range(CHUNK // L):
            slot_v[g2, pl.ds(k * L, L)] = neg1

    sup0 = wid * NSW
    nsw = jnp.minimum(NSW, jnp.maximum(NSUP - sup0, 0))

    def issue(sup, buf_slot, sem):
        @pl.when(sup == TAIL_SUP)
        def _():
            pltpu.async_copy(tail128_hbm,
                             blk_v.at[pl.ds(buf_slot * D, D),
                                      pl.ds(0, 128)], sem)

        @pl.when(sup != TAIL_SUP)
        def _():
            start = pl.multiple_of(sup * (SUP * 128), SUP * 128)
            pltpu.async_copy(entT_hbm.at[:, pl.ds(start, SUP * 128)],
                             blk_v.at[pl.ds(buf_slot * D, D), :], sem)

    def drain(sup, buf_slot, sem):
        @pl.when(sup == TAIL_SUP)
        def _():
            pltpu.make_async_copy(
                entT_hbm.at[:, pl.ds(0, 128)],
                blk_v.at[pl.ds(buf_slot * D, D), pl.ds(0, 128)],
                sem).wait()

        @pl.when(sup != TAIL_SUP)
        def _():
            pltpu.make_async_copy(
                entT_hbm.at[:, pl.ds(0, SUP * 128)],
                blk_v.at[pl.ds(buf_slot * D, D), :], sem).wait()

    def process_block(bl, buf_slot, q, sbc0):
        cntv = plsc.load_gather(counts_v, [_splat(bl)])
        cnt = _scalar(cntv)

        def entry_body(j2, sbc):
            ev = plsc.load_gather(bins_v, [_splat(bl * KBIN + j2)])
            e = _scalar(ev)
            c = (e & 127) + q * 128
            slot = lax.shift_right_logical(e, 7)
            r = sbc & (CHUNK - 1)
            grp = lax.shift_right_logical(sbc, 7) & 1
            row = grp * CHUNK + r
            for d in range(DL):
                vals = plsc.load_gather(
                    blk_v, [_splat(buf_slot * D + d * L) + lane, _splat(c)])
                sb_v[row, pl.ds(d * L, L)] = vals
            plsc.store_scatter(slot_v, [_splat(grp), _splat(r)],
                               _splat(slot), mask=lane0)

            @pl.when(r == CHUNK - 1)
            def _():
                cp = pltpu.async_copy(
                    sb_v.at[pl.ds(grp * CHUNK, CHUNK), :],
                    stage_hbm.at[plsc.Indices(slot_v.at[grp],
                                              ignored_value=-1)],
                    sem_s)
                cp.wait()
                for k in range(CHUNK // L):
                    slot_v[grp, pl.ds(k * L, L)] = neg1

            return sbc + 1

        return lax.fori_loop(0, cnt, entry_body, sbc0)

    def process_sup(sp, buf_slot, sbc):
        for q in range(SUP):
            bl = sp * SUP + q
            sbc = lax.cond(bl < nbw,
                           lambda s, bl=bl, q=q: process_block(
                               bl, buf_slot, q, s),
                           lambda s: s, sbc)
        return sbc

    @pl.when(nsw > 0)
    def _():
        issue(sup0, 0, sem_b0)

    npair = (nsw + 1) // 2

    def pair_body(p, sbc):
        sp0 = p * 2
        sp1 = sp0 + 1
        drain(sup0 + sp0, 0, sem_b0)

        @pl.when(sp1 < nsw)
        def _():
            issue(sup0 + sp1, 1, sem_b1)

        sbc = process_sup(sp0, 0, sbc)

        @pl.when(sp1 < nsw)
        def _():
            drain(sup0 + sp1, 1, sem_b1)

        @pl.when(sp0 + 2 < nsw)
        def _():
            issue(sup0 + sp0 + 2, 0, sem_b0)

        return lax.cond(sp1 < nsw,
                        lambda s: process_sup(sp1, 1, s),
                        lambda s: s, sbc)

    sb_cnt = lax.fori_loop(0, npair, pair_body, jnp.int32(0))

    @pl.when((sb_cnt & (CHUNK - 1)) != 0)
    def _():
        grp = lax.shift_right_logical(sb_cnt, 7) & 1
        cp = pltpu.async_copy(
            sb_v.at[pl.ds(grp * CHUNK, CHUNK), :],
            stage_hbm.at[plsc.Indices(slot_v.at[grp], ignored_value=-1)],
            sem_s)
        cp.wait()


def _score_body(rels_hbm, stage_hbm, rel128_hbm, out_hbm,
                ir_v, h_v, t_v, rn_v, scores_v, sem):
    wid = lax.axis_index("s") * NC + lax.axis_index("c")
    base = wid * BPW
    lane = _lane()

    def do_chunk(c, _):
        off = base + c * CHUNK
        pltpu.sync_copy(rels_hbm.at[pl.ds(off, CHUNK)], ir_v)
        cp_h = pltpu.async_copy(stage_hbm.at[pl.ds(off, CHUNK), :], h_v,
                                sem)
        cp_t = pltpu.async_copy(stage_hbm.at[pl.ds(B + off, CHUNK), :], t_v,
                                sem)
        cp_r = pltpu.async_copy(rel128_hbm.at[ir_v], rn_v, sem)
        cp_h.wait()
        cp_t.wait()
        cp_r.wait()

        def do_group(g, _):
            vec = jnp.zeros((L,), jnp.float32)
            for j in range(L):
                row = g * L + j
                h = [h_v[row, pl.ds(d * L, L)] for d in range(DL)]
                t = [t_v[row, pl.ds(d * L, L)] for d in range(DL)]
                r = [rn_v[row, pl.ds(d * L, L)] for d in range(DL)]
                n = [rn_v[row, pl.ds(D + d * L, L)] for d in range(DL)]
                dh = jnp.sum((h[0] * n[0] + h[1] * n[1])
                             + (h[2] * n[2] + h[3] * n[3]))
                dt = jnp.sum((t[0] * n[0] + t[1] * n[1])
                             + (t[2] * n[2] + t[3] * n[3]))
                parts = [jnp.abs((h[d] - t[d]) + r[d] + (dt - dh) * n[d])
                         for d in range(DL)]
                s = jnp.sum((parts[0] + parts[1]) + (parts[2] + parts[3]))
                vec = jnp.where(lane == j, s, vec)
            scores_v[pl.ds(c * CHUNK + g * L, L)] = vec
            return 0

        lax.fori_loop(0, GROUPS, do_group, 0)
        return 0

    lax.fori_loop(0, NCHUNK, do_chunk, 0)
    pltpu.sync_copy(scores_v, out_hbm.at[pl.ds(base, BPW)])


@jax.jit
def kernel(heads, rels, tails, ent_embs, rel_embs, norm_vector):
    entT = ent_embs.T
    tail128 = jnp.concatenate(
        [ent_embs[TAIL_BASE:].T,
         jnp.zeros((D, 128 - (NUM_ENT - TAIL_BASE)), jnp.float32)], axis=1)
    rel128 = jnp.concatenate([rel_embs, norm_vector], axis=1)
    mesh = plsc.VectorSubcoreMesh(core_axis_name="c", subcore_axis_name="s",
                                  num_cores=NC, num_subcores=NS)
    cp = pltpu.CompilerParams(needs_layout_passes=False,
                              use_tc_tiling_on_sc=True)

    stage = pl.kernel(
        _extract_body,
        out_type=jax.ShapeDtypeStruct((2 * B, 128), jnp.float32),
        mesh=mesh,
        compiler_params=cp,
        scratch_types=[
            pltpu.VMEM((SCAN,), jnp.int32),
            pltpu.VMEM((COMP,), jnp.int32),
            pltpu.VMEM((NBW_PAD,), jnp.int32),
            pltpu.VMEM((NBW_PAD * KBIN,), jnp.int32),
            pltpu.VMEM((2 * D, SUP * 128), jnp.float32),
            pltpu.VMEM((SB_ROWS, 128), jnp.float32),
            pltpu.VMEM((2, CHUNK), jnp.int32),
            pltpu.SemaphoreType.DMA,
            pltpu.SemaphoreType.DMA,
            pltpu.SemaphoreType.DMA,
        ],
    )(heads, tails, entT, tail128)

    scores = pl.kernel(
        _score_body,
        out_type=jax.ShapeDtypeStruct((B,), jnp.float32),
        mesh=mesh,
        compiler_params=cp,
        scratch_types=[
            pltpu.VMEM((CHUNK,), jnp.int32),
            pltpu.VMEM((CHUNK, 128), jnp.float32),
            pltpu.VMEM((CHUNK, 128), jnp.float32),
            pltpu.VMEM((CHUNK, 128), jnp.float32),
            pltpu.VMEM((BPW,), jnp.float32),
            pltpu.SemaphoreType.DMA,
        ],
    )(rels, stage, rel128)
    return scores
